# Initial kernel scaffold; baseline (speedup 1.0000x reference)
import jax
import jax.numpy as jnp
from jax.experimental import pallas as pl


def kernel(x, edge_index, W1, b1, W2, b2):
    raise NotImplementedError("write your pallas kernel here")


# speedup vs baseline: 9.5938x; 9.5938x over previous
import functools

import jax
import jax.numpy as jnp
from jax import lax
from jax.experimental import pallas as pl
from jax.experimental.pallas import tpu as pltpu
from jax.experimental.pallas import tpu_sc as plsc

N = 10000
E = 320000
CIN = 128
CH = 64
HID = 256
K = 10
NCORE = 2
NTILE = 16
BB = 128
NK = 157
EPAD = NTILE * NK * BB
NPR = 640
NP = NTILE * NPR
UC = 128
NU = NPR // UC
BM = 400


def _mesh():
    return plsc.VectorSubcoreMesh(
        core_axis_name="c", subcore_axis_name="s",
        num_cores=NCORE, num_subcores=NTILE)


def _fill(ref, rows, value):
    width = ref.shape[1]
    val = jnp.full((16,), value, jnp.float32)

    @pl.loop(0, rows)
    def _(n):
        for j in range(width // 16):
            ref[n, pl.ds(j * 16, 16)] = val


def _deg_body(dstI, deg3d, dst_v, ones_v, zero_v, stage_v, deg_sp):
    c = lax.axis_index("c")

    @pl.when(c == 0)
    def _():
        w = lax.axis_index("s")
        _fill(ones_v, BB, 1.0)
        _fill(zero_v, UC, 0.0)
        pltpu.sync_copy(dstI.at[w], dst_v)
        for i in range(NU):
            pltpu.sync_copy(zero_v, deg_sp.at[pl.ds(w * NPR + i * UC, UC)])
        plsc.subcore_barrier()

        @pl.loop(0, NK)
        def _(j):
            pltpu.sync_copy(ones_v, deg_sp.at[dst_v.at[j]], add=True)

        plsc.subcore_barrier()
        pltpu.sync_copy(deg_sp.at[pl.ds(w * NPR, NPR)], stage_v)
        pltpu.sync_copy(stage_v, deg3d.at[w])


def _make_deg_kernel():
    return pl.kernel(
        _deg_body,
        out_type=jax.ShapeDtypeStruct((NTILE, NPR, 16), jnp.float32),
        mesh=_mesh(),
        compiler_params=pltpu.CompilerParams(use_tc_tiling_on_sc=False),
        scratch_types=[
            pltpu.VMEM((NK, BB), jnp.int32),
            pltpu.VMEM((BB, 16), jnp.float32),
            pltpu.VMEM((UC, 16), jnp.float32),
            pltpu.VMEM((NPR, 16), jnp.float32),
            pltpu.VMEM_SHARED((NP, 16), jnp.float32),
        ],
    )


def _iter_body(gl, gh, srcI, dstI, a2d, cl, ch, gol, goh,
               src_v, dst_v, rows_v, a_v, st_a, st_c,
               agg_sp, gsem):
    c = lax.axis_index("c")
    w = lax.axis_index("s")

    _fill(st_a, UC, 0.0)
    pltpu.sync_copy(srcI.at[w], src_v)
    pltpu.sync_copy(dstI.at[w], dst_v)
    pltpu.sync_copy(a2d.at[w], a_v)
    for i in range(NU):
        pltpu.sync_copy(st_a, agg_sp.at[pl.ds(w * NPR + i * UC, UC)])
    plsc.subcore_barrier()

    def edge_phase(g_in):
        @pl.loop(0, NK)
        def _(j):
            pltpu.async_copy(g_in.at[src_v.at[j]], rows_v, gsem).wait()
            pltpu.sync_copy(rows_v, agg_sp.at[dst_v.at[j]], add=True)

    @pl.when(c == 0)
    def _():
        edge_phase(gl)

    @pl.when(c == 1)
    def _():
        edge_phase(gh)

    plsc.subcore_barrier()

    def update_phase(g_in, c_in, g_out):
        @pl.loop(0, NU)
        def _(u):
            base = w * NPR + u * UC
            pltpu.sync_copy(agg_sp.at[pl.ds(base, UC)], st_a)
            pltpu.sync_copy(g_in.at[pl.ds(base, UC)], rows_v)
            pltpu.sync_copy(c_in.at[pl.ds(base, UC)], st_c)

            @pl.loop(0, UC)
            def _(n):
                av = plsc.load_gather(
                    a_v, [jnp.full((16,), u * UC + n, jnp.int32)])
                for jj in range(CH // 16):
                    sl = pl.ds(jj * 16, 16)
                    st_a[n, sl] = av * (st_a[n, sl] + rows_v[n, sl]) + st_c[n, sl]

            pltpu.sync_copy(st_a, g_out.at[pl.ds(base, UC)])

    @pl.when(c == 0)
    def _():
        update_phase(gl, cl, gol)

    @pl.when(c == 1)
    def _():
        update_phase(gh, ch, goh)


def _make_iter_kernel():
    return pl.kernel(
        _iter_body,
        out_type=(jax.ShapeDtypeStruct((NP, CH), jnp.float32),
                  jax.ShapeDtypeStruct((NP, CH), jnp.float32)),
        mesh=_mesh(),
        compiler_params=pltpu.CompilerParams(
            use_tc_tiling_on_sc=False, needs_layout_passes=False),
        scratch_types=[
            pltpu.VMEM((NK, BB), jnp.int32),
            pltpu.VMEM((NK, BB), jnp.int32),
            pltpu.VMEM((BB, CH), jnp.float32),
            pltpu.VMEM((NPR,), jnp.float32),
            pltpu.VMEM((UC, CH), jnp.float32),
            pltpu.VMEM((UC, CH), jnp.float32),
            pltpu.VMEM_SHARED((NP, CH), jnp.float32),
            pltpu.SemaphoreType.DMA,
        ],
    )


def _mlp_body(h_ref, w1_ref, b1_ref, w2_ref, b2_ref, o_ref):
    hb = h_ref[...]
    t = jnp.dot(hb, w1_ref[...], preferred_element_type=jnp.float32,
                precision=lax.Precision.HIGHEST)
    t = jnp.maximum(t + b1_ref[...], 0.0)
    o = jnp.dot(t, w2_ref[...], preferred_element_type=jnp.float32,
                precision=lax.Precision.HIGHEST)
    o_ref[...] = o + b2_ref[...]


def _mlp(h, W1, b1, W2, b2):
    return pl.pallas_call(
        _mlp_body,
        grid=(N // BM,),
        in_specs=[
            pl.BlockSpec((BM, CIN), lambda i: (i, 0)),
            pl.BlockSpec((CIN, HID), lambda i: (0, 0)),
            pl.BlockSpec((1, HID), lambda i: (0, 0)),
            pl.BlockSpec((HID, CIN), lambda i: (0, 0)),
            pl.BlockSpec((1, CIN), lambda i: (0, 0)),
        ],
        out_specs=pl.BlockSpec((BM, CIN), lambda i: (i, 0)),
        out_shape=jax.ShapeDtypeStruct((N, CIN), jnp.float32),
    )(h, W1, b1.reshape(1, HID), W2, b2.reshape(1, CIN))


def kernel(x, edge_index, W1, b1, W2, b2):
    src = edge_index[0].astype(jnp.int32)
    dst = edge_index[1].astype(jnp.int32)
    srcI = jnp.concatenate(
        [src, jnp.zeros((EPAD - E,), jnp.int32)]).reshape(NTILE, NK, BB)
    dstI = jnp.concatenate(
        [dst, jnp.full((EPAD - E,), N, jnp.int32)]).reshape(NTILE, NK, BB)

    deg3d = _make_deg_kernel()(dstI)
    deg = deg3d[:, :, 0].reshape(NP)[:N] + 1.0
    dinv = lax.rsqrt(jnp.maximum(deg, 1e-12))

    def pad1(v):
        return jnp.pad(v, (0, NP - N)).reshape(NTILE, NPR)

    def pad2(m):
        return jnp.pad(m, ((0, NP - N), (0, 0)))

    a_mid = pad1(0.9 * dinv * dinv)
    a_fin = pad1(0.9 * dinv)
    cm = pad2(0.1 * dinv[:, None] * x)
    cf = pad2(0.1 * x)
    g0 = pad2(dinv[:, None] * x)

    gl, gh = g0[:, :CH], g0[:, CH:]
    cml, cmh = cm[:, :CH], cm[:, CH:]
    cfl, cfh = cf[:, :CH], cf[:, CH:]

    it = _make_iter_kernel()
    for k in range(K):
        if k < K - 1:
            gl, gh = it(gl, gh, srcI, dstI, a_mid, cml, cmh)
        else:
            gl, gh = it(gl, gh, srcI, dstI, a_fin, cfl, cfh)

    h = jnp.concatenate([gl[:N], gh[:N]], axis=1)
    return _mlp(h, W1, b1, W2, b2)


# speedup vs baseline: 11.8164x; 1.2317x over previous
import functools

import jax
import jax.numpy as jnp
from jax import lax
from jax.experimental import pallas as pl
from jax.experimental.pallas import tpu as pltpu
from jax.experimental.pallas import tpu_sc as plsc

N = 10000
E = 320000
CIN = 128
CH = 64
HID = 256
K = 10
NCORE = 2
NTILE = 16
BB = 128
NK = 157
EPAD = NTILE * NK * BB
NPR = 640
NP = NTILE * NPR
UC = 128
NU = NPR // UC
BM = 400


def _mesh():
    return plsc.VectorSubcoreMesh(
        core_axis_name="c", subcore_axis_name="s",
        num_cores=NCORE, num_subcores=NTILE)


def _fill(ref, rows, value):
    width = ref.shape[1]
    val = jnp.full((16,), value, jnp.float32)

    @pl.loop(0, rows)
    def _(n):
        for j in range(width // 16):
            ref[n, pl.ds(j * 16, 16)] = val


def _deg_body(dstI, deg3d, dst_v, ones_v, zero_v, stage_v, deg_sp):
    c = lax.axis_index("c")

    @pl.when(c == 0)
    def _():
        w = lax.axis_index("s")
        _fill(ones_v, BB, 1.0)
        _fill(zero_v, UC, 0.0)
        pltpu.sync_copy(dstI.at[w], dst_v)
        for i in range(NU):
            pltpu.sync_copy(zero_v, deg_sp.at[pl.ds(w * NPR + i * UC, UC)])
        plsc.subcore_barrier()

        @pl.loop(0, NK)
        def _(j):
            pltpu.sync_copy(ones_v, deg_sp.at[dst_v.at[j]], add=True)

        plsc.subcore_barrier()
        pltpu.sync_copy(deg_sp.at[pl.ds(w * NPR, NPR)], stage_v)
        pltpu.sync_copy(stage_v, deg3d.at[w])


def _make_deg_kernel():
    return pl.kernel(
        _deg_body,
        out_type=jax.ShapeDtypeStruct((NTILE, NPR, 16), jnp.float32),
        mesh=_mesh(),
        compiler_params=pltpu.CompilerParams(use_tc_tiling_on_sc=False),
        scratch_types=[
            pltpu.VMEM((NK, BB), jnp.int32),
            pltpu.VMEM((BB, 16), jnp.float32),
            pltpu.VMEM((UC, 16), jnp.float32),
            pltpu.VMEM((NPR, 16), jnp.float32),
            pltpu.VMEM_SHARED((NP, 16), jnp.float32),
        ],
    )


def _iter_body(gl, gh, srcI, dstI, a2d, cl, ch, gol, goh,
               src_v, dst_v, rows_v, a_v, st_a, st_c,
               agg_sp, gsem):
    c = lax.axis_index("c")
    w = lax.axis_index("s")

    _fill(st_a, UC, 0.0)
    pltpu.sync_copy(srcI.at[w], src_v)
    pltpu.sync_copy(dstI.at[w], dst_v)
    pltpu.sync_copy(a2d.at[w], a_v)
    for i in range(NU):
        pltpu.sync_copy(st_a, agg_sp.at[pl.ds(w * NPR + i * UC, UC)])
    plsc.subcore_barrier()

    def edge_phase(g_in):
        pltpu.async_copy(g_in.at[src_v.at[0]], rows_v.at[0], gsem)

        @pl.loop(0, NK)
        def _(j):
            b = lax.rem(j, 2)
            pltpu.make_async_copy(
                g_in.at[src_v.at[j]], rows_v.at[b], gsem).wait()

            @pl.when(j + 1 < NK)
            def _():
                pltpu.async_copy(
                    g_in.at[src_v.at[j + 1]], rows_v.at[1 - b], gsem)

            pltpu.sync_copy(rows_v.at[b], agg_sp.at[dst_v.at[j]], add=True)

    @pl.when(c == 0)
    def _():
        edge_phase(gl)

    @pl.when(c == 1)
    def _():
        edge_phase(gh)

    plsc.subcore_barrier()

    def update_phase(g_in, c_in, g_out):
        @pl.loop(0, NU)
        def _(u):
            base = w * NPR + u * UC
            pltpu.sync_copy(agg_sp.at[pl.ds(base, UC)], st_a)
            pltpu.sync_copy(g_in.at[pl.ds(base, UC)], rows_v.at[0])
            pltpu.sync_copy(c_in.at[pl.ds(base, UC)], st_c)

            @pl.loop(0, UC)
            def _(n):
                av = plsc.load_gather(
                    a_v, [jnp.full((16,), u * UC + n, jnp.int32)])
                for jj in range(CH // 16):
                    sl = pl.ds(jj * 16, 16)
                    st_a[n, sl] = (av * (st_a[n, sl] + rows_v[0, n, sl])
                                   + st_c[n, sl])

            pltpu.sync_copy(st_a, g_out.at[pl.ds(base, UC)])

    @pl.when(c == 0)
    def _():
        update_phase(gl, cl, gol)

    @pl.when(c == 1)
    def _():
        update_phase(gh, ch, goh)


def _make_iter_kernel():
    return pl.kernel(
        _iter_body,
        out_type=(jax.ShapeDtypeStruct((NP, CH), jnp.float32),
                  jax.ShapeDtypeStruct((NP, CH), jnp.float32)),
        mesh=_mesh(),
        compiler_params=pltpu.CompilerParams(
            use_tc_tiling_on_sc=False, needs_layout_passes=False),
        scratch_types=[
            pltpu.VMEM((NK, BB), jnp.int32),
            pltpu.VMEM((NK, BB), jnp.int32),
            pltpu.VMEM((2, BB, CH), jnp.float32),
            pltpu.VMEM((NPR,), jnp.float32),
            pltpu.VMEM((UC, CH), jnp.float32),
            pltpu.VMEM((UC, CH), jnp.float32),
            pltpu.VMEM_SHARED((NP, CH), jnp.float32),
            pltpu.SemaphoreType.DMA,
        ],
    )


def _mlp_body(h_ref, w1_ref, b1_ref, w2_ref, b2_ref, o_ref):
    hb = h_ref[...]
    t = jnp.dot(hb, w1_ref[...], preferred_element_type=jnp.float32,
                precision=lax.Precision.HIGHEST)
    t = jnp.maximum(t + b1_ref[...], 0.0)
    o = jnp.dot(t, w2_ref[...], preferred_element_type=jnp.float32,
                precision=lax.Precision.HIGHEST)
    o_ref[...] = o + b2_ref[...]


def _mlp(h, W1, b1, W2, b2):
    return pl.pallas_call(
        _mlp_body,
        grid=(N // BM,),
        in_specs=[
            pl.BlockSpec((BM, CIN), lambda i: (i, 0)),
            pl.BlockSpec((CIN, HID), lambda i: (0, 0)),
            pl.BlockSpec((1, HID), lambda i: (0, 0)),
            pl.BlockSpec((HID, CIN), lambda i: (0, 0)),
            pl.BlockSpec((1, CIN), lambda i: (0, 0)),
        ],
        out_specs=pl.BlockSpec((BM, CIN), lambda i: (i, 0)),
        out_shape=jax.ShapeDtypeStruct((N, CIN), jnp.float32),
    )(h, W1, b1.reshape(1, HID), W2, b2.reshape(1, CIN))


def kernel(x, edge_index, W1, b1, W2, b2):
    src = edge_index[0].astype(jnp.int32)
    dst = edge_index[1].astype(jnp.int32)
    srcI = jnp.concatenate(
        [src, jnp.zeros((EPAD - E,), jnp.int32)]).reshape(NTILE, NK, BB)
    dstI = jnp.concatenate(
        [dst, jnp.full((EPAD - E,), N, jnp.int32)]).reshape(NTILE, NK, BB)

    deg3d = _make_deg_kernel()(dstI)
    deg = deg3d[:, :, 0].reshape(NP)[:N] + 1.0
    dinv = lax.rsqrt(jnp.maximum(deg, 1e-12))

    def pad1(v):
        return jnp.pad(v, (0, NP - N)).reshape(NTILE, NPR)

    def pad2(m):
        return jnp.pad(m, ((0, NP - N), (0, 0)))

    a_mid = pad1(0.9 * dinv * dinv)
    a_fin = pad1(0.9 * dinv)
    cm = pad2(0.1 * dinv[:, None] * x)
    cf = pad2(0.1 * x)
    g0 = pad2(dinv[:, None] * x)

    gl, gh = g0[:, :CH], g0[:, CH:]
    cml, cmh = cm[:, :CH], cm[:, CH:]
    cfl, cfh = cf[:, :CH], cf[:, CH:]

    it = _make_iter_kernel()
    for k in range(K):
        if k < K - 1:
            gl, gh = it(gl, gh, srcI, dstI, a_mid, cml, cmh)
        else:
            gl, gh = it(gl, gh, srcI, dstI, a_fin, cfl, cfh)

    h = jnp.concatenate([gl[:N], gh[:N]], axis=1)
    return _mlp(h, W1, b1, W2, b2)


# speedup vs baseline: 16.0358x; 1.3571x over previous
import functools

import jax
import jax.numpy as jnp
from jax import lax
from jax.experimental import pallas as pl
from jax.experimental.pallas import tpu as pltpu
from jax.experimental.pallas import tpu_sc as plsc

N = 10000
E = 320000
CIN = 128
CH = 64
HID = 256
K = 10
NCORE = 2
NTILE = 16
BB = 128
NK = 157
EPAD = NTILE * NK * BB
NPR = 640
NP = NTILE * NPR
UC = 128
NU = NPR // UC
BM = 400


def _mesh():
    return plsc.VectorSubcoreMesh(
        core_axis_name="c", subcore_axis_name="s",
        num_cores=NCORE, num_subcores=NTILE)


def _fill(ref, rows, value):
    width = ref.shape[1]
    val = jnp.full((16,), value, jnp.float32)

    @pl.loop(0, rows)
    def _(n):
        for j in range(width // 16):
            ref[n, pl.ds(j * 16, 16)] = val


def _deg_body(dstI, deg3d, dst_v, ones_v, zero_v, stage_v, deg_sp):
    c = lax.axis_index("c")

    @pl.when(c == 0)
    def _():
        w = lax.axis_index("s")
        _fill(ones_v, BB, 1.0)
        _fill(zero_v, UC, 0.0)
        pltpu.sync_copy(dstI.at[w], dst_v)
        for i in range(NU):
            pltpu.sync_copy(zero_v, deg_sp.at[pl.ds(w * NPR + i * UC, UC)])
        plsc.subcore_barrier()

        @pl.loop(0, NK)
        def _(j):
            pltpu.sync_copy(ones_v, deg_sp.at[dst_v.at[j]], add=True)

        plsc.subcore_barrier()
        pltpu.sync_copy(deg_sp.at[pl.ds(w * NPR, NPR)], stage_v)
        pltpu.sync_copy(stage_v, deg3d.at[w])


def _make_deg_kernel():
    return pl.kernel(
        _deg_body,
        out_type=jax.ShapeDtypeStruct((NTILE, NPR, 16), jnp.float32),
        mesh=_mesh(),
        compiler_params=pltpu.CompilerParams(use_tc_tiling_on_sc=False),
        scratch_types=[
            pltpu.VMEM((NK, BB), jnp.int32),
            pltpu.VMEM((BB, 16), jnp.float32),
            pltpu.VMEM((UC, 16), jnp.float32),
            pltpu.VMEM((NPR, 16), jnp.float32),
            pltpu.VMEM_SHARED((NP, 16), jnp.float32),
        ],
    )


def _iter_body(gl, gh, srcI, dstI, a2d, cl, ch, gol, goh,
               src_v, dst_v, rows_v, a_v, st_a, st_c,
               agg_sp, gsem, ssem):
    c = lax.axis_index("c")
    w = lax.axis_index("s")

    _fill(st_a, UC, 0.0)
    pltpu.sync_copy(srcI.at[w], src_v)
    pltpu.sync_copy(dstI.at[w], dst_v)
    pltpu.sync_copy(a2d.at[w], a_v)
    for i in range(NU):
        pltpu.sync_copy(st_a, agg_sp.at[pl.ds(w * NPR + i * UC, UC)])
    plsc.subcore_barrier()

    def edge_phase(g_in):
        pltpu.async_copy(g_in.at[src_v.at[0]], rows_v.at[0], gsem)
        pltpu.async_copy(g_in.at[src_v.at[1]], rows_v.at[1], gsem)

        @pl.loop(0, NK)
        def _(j):
            b = lax.rem(j, 4)
            pltpu.make_async_copy(
                g_in.at[src_v.at[j]], rows_v.at[b], gsem).wait()
            pltpu.async_copy(rows_v.at[b], agg_sp.at[dst_v.at[j]], ssem,
                             add=True)

            @pl.when(j >= 1)
            def _():
                pltpu.make_async_copy(
                    rows_v.at[b], agg_sp.at[dst_v.at[j]], ssem).wait()

            @pl.when(j + 2 < NK)
            def _():
                pltpu.async_copy(g_in.at[src_v.at[j + 2]],
                                 rows_v.at[lax.rem(j + 2, 4)], gsem)

        pltpu.make_async_copy(
            rows_v.at[0], agg_sp.at[dst_v.at[0]], ssem).wait()

    @pl.when(c == 0)
    def _():
        edge_phase(gl)

    @pl.when(c == 1)
    def _():
        edge_phase(gh)

    plsc.subcore_barrier()

    def update_phase(g_in, c_in, g_out):
        @pl.loop(0, NU)
        def _(u):
            base = w * NPR + u * UC
            pltpu.sync_copy(agg_sp.at[pl.ds(base, UC)], st_a)
            pltpu.sync_copy(g_in.at[pl.ds(base, UC)], rows_v.at[0])
            pltpu.sync_copy(c_in.at[pl.ds(base, UC)], st_c)

            @pl.loop(0, UC)
            def _(n):
                av = plsc.load_gather(
                    a_v, [jnp.full((16,), u * UC + n, jnp.int32)])
                for jj in range(CH // 16):
                    sl = pl.ds(jj * 16, 16)
                    st_a[n, sl] = (av * (st_a[n, sl] + rows_v[0, n, sl])
                                   + st_c[n, sl])

            pltpu.sync_copy(st_a, g_out.at[pl.ds(base, UC)])

    @pl.when(c == 0)
    def _():
        update_phase(gl, cl, gol)

    @pl.when(c == 1)
    def _():
        update_phase(gh, ch, goh)


def _make_iter_kernel():
    return pl.kernel(
        _iter_body,
        out_type=(jax.ShapeDtypeStruct((NP, CH), jnp.float32),
                  jax.ShapeDtypeStruct((NP, CH), jnp.float32)),
        mesh=_mesh(),
        compiler_params=pltpu.CompilerParams(
            use_tc_tiling_on_sc=False, needs_layout_passes=False),
        scratch_types=[
            pltpu.VMEM((NK, BB), jnp.int32),
            pltpu.VMEM((NK, BB), jnp.int32),
            pltpu.VMEM((4, BB, CH), jnp.float32),
            pltpu.VMEM((NPR,), jnp.float32),
            pltpu.VMEM((UC, CH), jnp.float32),
            pltpu.VMEM((UC, CH), jnp.float32),
            pltpu.VMEM_SHARED((NP, CH), jnp.float32),
            pltpu.SemaphoreType.DMA,
            pltpu.SemaphoreType.DMA,
        ],
    )


def _mlp_body(h_ref, w1_ref, b1_ref, w2_ref, b2_ref, o_ref):
    hb = h_ref[...]
    t = jnp.dot(hb, w1_ref[...], preferred_element_type=jnp.float32,
                precision=lax.Precision.HIGHEST)
    t = jnp.maximum(t + b1_ref[...], 0.0)
    o = jnp.dot(t, w2_ref[...], preferred_element_type=jnp.float32,
                precision=lax.Precision.HIGHEST)
    o_ref[...] = o + b2_ref[...]


def _mlp(h, W1, b1, W2, b2):
    return pl.pallas_call(
        _mlp_body,
        grid=(N // BM,),
        in_specs=[
            pl.BlockSpec((BM, CIN), lambda i: (i, 0)),
            pl.BlockSpec((CIN, HID), lambda i: (0, 0)),
            pl.BlockSpec((1, HID), lambda i: (0, 0)),
            pl.BlockSpec((HID, CIN), lambda i: (0, 0)),
            pl.BlockSpec((1, CIN), lambda i: (0, 0)),
        ],
        out_specs=pl.BlockSpec((BM, CIN), lambda i: (i, 0)),
        out_shape=jax.ShapeDtypeStruct((N, CIN), jnp.float32),
    )(h, W1, b1.reshape(1, HID), W2, b2.reshape(1, CIN))


def kernel(x, edge_index, W1, b1, W2, b2):
    src = edge_index[0].astype(jnp.int32)
    dst = edge_index[1].astype(jnp.int32)
    srcI = jnp.concatenate(
        [src, jnp.zeros((EPAD - E,), jnp.int32)]).reshape(NTILE, NK, BB)
    dstI = jnp.concatenate(
        [dst, jnp.full((EPAD - E,), N, jnp.int32)]).reshape(NTILE, NK, BB)

    deg3d = _make_deg_kernel()(dstI)
    deg = deg3d[:, :, 0].reshape(NP)[:N] + 1.0
    dinv = lax.rsqrt(jnp.maximum(deg, 1e-12))

    def pad1(v):
        return jnp.pad(v, (0, NP - N)).reshape(NTILE, NPR)

    def pad2(m):
        return jnp.pad(m, ((0, NP - N), (0, 0)))

    a_mid = pad1(0.9 * dinv * dinv)
    a_fin = pad1(0.9 * dinv)
    cm = pad2(0.1 * dinv[:, None] * x)
    cf = pad2(0.1 * x)
    g0 = pad2(dinv[:, None] * x)

    gl, gh = g0[:, :CH], g0[:, CH:]
    cml, cmh = cm[:, :CH], cm[:, CH:]
    cfl, cfh = cf[:, :CH], cf[:, CH:]

    it = _make_iter_kernel()
    for k in range(K):
        if k < K - 1:
            gl, gh = it(gl, gh, srcI, dstI, a_mid, cml, cmh)
        else:
            gl, gh = it(gl, gh, srcI, dstI, a_fin, cfl, cfh)

    h = jnp.concatenate([gl[:N], gh[:N]], axis=1)
    return _mlp(h, W1, b1, W2, b2)


# speedup vs baseline: 16.7224x; 1.0428x over previous
import functools

import jax
import jax.numpy as jnp
from jax import lax
from jax.experimental import pallas as pl
from jax.experimental.pallas import tpu as pltpu
from jax.experimental.pallas import tpu_sc as plsc

N = 10000
E = 320000
CIN = 128
CH = 64
HID = 256
K = 10
NCORE = 2
NTILE = 16
BB = 128
NK = 157
EPAD = NTILE * NK * BB
NPR = 640
NP = NTILE * NPR
UC = 128
NU = NPR // UC
BM = 400


def _mesh():
    return plsc.VectorSubcoreMesh(
        core_axis_name="c", subcore_axis_name="s",
        num_cores=NCORE, num_subcores=NTILE)


def _fill(ref, rows, value):
    width = ref.shape[1]
    val = jnp.full((16,), value, jnp.float32)

    @pl.loop(0, rows)
    def _(n):
        for j in range(width // 16):
            ref[n, pl.ds(j * 16, 16)] = val


def _deg_body(dstI, deg3d, dst_v, ones_v, zero_v, stage_v, deg_sp):
    c = lax.axis_index("c")

    @pl.when(c == 0)
    def _():
        w = lax.axis_index("s")
        _fill(ones_v, BB, 1.0)
        _fill(zero_v, UC, 0.0)
        pltpu.sync_copy(dstI.at[w], dst_v)
        for i in range(NU):
            pltpu.sync_copy(zero_v, deg_sp.at[pl.ds(w * NPR + i * UC, UC)])
        plsc.subcore_barrier()

        @pl.loop(0, NK)
        def _(j):
            pltpu.sync_copy(ones_v, deg_sp.at[dst_v.at[j]], add=True)

        plsc.subcore_barrier()
        pltpu.sync_copy(deg_sp.at[pl.ds(w * NPR, NPR)], stage_v)
        pltpu.sync_copy(stage_v, deg3d.at[w])


def _make_deg_kernel():
    return pl.kernel(
        _deg_body,
        out_type=jax.ShapeDtypeStruct((NTILE, NPR, 16), jnp.float32),
        mesh=_mesh(),
        compiler_params=pltpu.CompilerParams(use_tc_tiling_on_sc=False),
        scratch_types=[
            pltpu.VMEM((NK, BB), jnp.int32),
            pltpu.VMEM((BB, 16), jnp.float32),
            pltpu.VMEM((UC, 16), jnp.float32),
            pltpu.VMEM((NPR, 16), jnp.float32),
            pltpu.VMEM_SHARED((NP, 16), jnp.float32),
        ],
    )


def _iter_body(gl, gh, srcI, dstI, a2d, cl, ch, gol, goh,
               src_v, dst_v, rows_v, a_v, st_a, st_c,
               agg_sp, gsem, ssem):
    c = lax.axis_index("c")
    w = lax.axis_index("s")

    _fill(st_a, UC, 0.0)
    pltpu.sync_copy(srcI.at[w], src_v)
    pltpu.sync_copy(dstI.at[w], dst_v)
    pltpu.sync_copy(a2d.at[w], a_v)
    for i in range(NU):
        pltpu.sync_copy(st_a, agg_sp.at[pl.ds(w * NPR + i * UC, UC)])
    plsc.subcore_barrier()

    def edge_phase(g_in):
        pltpu.async_copy(g_in.at[src_v.at[0]], rows_v.at[0], gsem)
        pltpu.async_copy(g_in.at[src_v.at[1]], rows_v.at[1], gsem)
        pltpu.async_copy(g_in.at[src_v.at[2]], rows_v.at[2], gsem)

        @pl.loop(0, NK)
        def _(j):
            b = lax.rem(j, 4)
            pltpu.make_async_copy(
                g_in.at[src_v.at[j]], rows_v.at[b], gsem).wait()
            pltpu.async_copy(rows_v.at[b], agg_sp.at[dst_v.at[j]], ssem,
                             add=True)

            @pl.when(j >= 1)
            def _():
                pltpu.make_async_copy(
                    rows_v.at[b], agg_sp.at[dst_v.at[j]], ssem).wait()

            @pl.when(j + 3 < NK)
            def _():
                pltpu.async_copy(g_in.at[src_v.at[j + 3]],
                                 rows_v.at[lax.rem(j + 3, 4)], gsem)

        pltpu.make_async_copy(
            rows_v.at[0], agg_sp.at[dst_v.at[0]], ssem).wait()

    @pl.when(c == 0)
    def _():
        edge_phase(gl)

    @pl.when(c == 1)
    def _():
        edge_phase(gh)

    plsc.subcore_barrier()

    def update_phase(g_in, c_in, g_out):
        @pl.loop(0, NU)
        def _(u):
            base = w * NPR + u * UC
            pltpu.sync_copy(agg_sp.at[pl.ds(base, UC)], st_a)
            pltpu.sync_copy(g_in.at[pl.ds(base, UC)], rows_v.at[0])
            pltpu.sync_copy(c_in.at[pl.ds(base, UC)], st_c)

            @pl.loop(0, UC)
            def _(n):
                av = plsc.load_gather(
                    a_v, [jnp.full((16,), u * UC + n, jnp.int32)])
                for jj in range(CH // 16):
                    sl = pl.ds(jj * 16, 16)
                    st_a[n, sl] = (av * (st_a[n, sl] + rows_v[0, n, sl])
                                   + st_c[n, sl])

            pltpu.sync_copy(st_a, g_out.at[pl.ds(base, UC)])

    @pl.when(c == 0)
    def _():
        update_phase(gl, cl, gol)

    @pl.when(c == 1)
    def _():
        update_phase(gh, ch, goh)


def _make_iter_kernel():
    return pl.kernel(
        _iter_body,
        out_type=(jax.ShapeDtypeStruct((NP, CH), jnp.float32),
                  jax.ShapeDtypeStruct((NP, CH), jnp.float32)),
        mesh=_mesh(),
        compiler_params=pltpu.CompilerParams(
            use_tc_tiling_on_sc=False, needs_layout_passes=False),
        scratch_types=[
            pltpu.VMEM((NK, BB), jnp.int32),
            pltpu.VMEM((NK, BB), jnp.int32),
            pltpu.VMEM((4, BB, CH), jnp.float32),
            pltpu.VMEM((NPR,), jnp.float32),
            pltpu.VMEM((UC, CH), jnp.float32),
            pltpu.VMEM((UC, CH), jnp.float32),
            pltpu.VMEM_SHARED((NP, CH), jnp.float32),
            pltpu.SemaphoreType.DMA,
            pltpu.SemaphoreType.DMA,
        ],
    )


def _mlp_body(h_ref, w1_ref, b1_ref, w2_ref, b2_ref, o_ref):
    hb = h_ref[...]
    t = jnp.dot(hb, w1_ref[...], preferred_element_type=jnp.float32,
                precision=lax.Precision.HIGHEST)
    t = jnp.maximum(t + b1_ref[...], 0.0)
    o = jnp.dot(t, w2_ref[...], preferred_element_type=jnp.float32,
                precision=lax.Precision.HIGHEST)
    o_ref[...] = o + b2_ref[...]


def _mlp(h, W1, b1, W2, b2):
    return pl.pallas_call(
        _mlp_body,
        grid=(N // BM,),
        in_specs=[
            pl.BlockSpec((BM, CIN), lambda i: (i, 0)),
            pl.BlockSpec((CIN, HID), lambda i: (0, 0)),
            pl.BlockSpec((1, HID), lambda i: (0, 0)),
            pl.BlockSpec((HID, CIN), lambda i: (0, 0)),
            pl.BlockSpec((1, CIN), lambda i: (0, 0)),
        ],
        out_specs=pl.BlockSpec((BM, CIN), lambda i: (i, 0)),
        out_shape=jax.ShapeDtypeStruct((N, CIN), jnp.float32),
    )(h, W1, b1.reshape(1, HID), W2, b2.reshape(1, CIN))


def kernel(x, edge_index, W1, b1, W2, b2):
    src = edge_index[0].astype(jnp.int32)
    dst = edge_index[1].astype(jnp.int32)
    srcI = jnp.concatenate(
        [src, jnp.zeros((EPAD - E,), jnp.int32)]).reshape(NTILE, NK, BB)
    dstI = jnp.concatenate(
        [dst, jnp.full((EPAD - E,), N, jnp.int32)]).reshape(NTILE, NK, BB)

    deg3d = _make_deg_kernel()(dstI)
    deg = deg3d[:, :, 0].reshape(NP)[:N] + 1.0
    dinv = lax.rsqrt(jnp.maximum(deg, 1e-12))

    def pad1(v):
        return jnp.pad(v, (0, NP - N)).reshape(NTILE, NPR)

    def pad2(m):
        return jnp.pad(m, ((0, NP - N), (0, 0)))

    a_mid = pad1(0.9 * dinv * dinv)
    a_fin = pad1(0.9 * dinv)
    cm = pad2(0.1 * dinv[:, None] * x)
    cf = pad2(0.1 * x)
    g0 = pad2(dinv[:, None] * x)

    gl, gh = g0[:, :CH], g0[:, CH:]
    cml, cmh = cm[:, :CH], cm[:, CH:]
    cfl, cfh = cf[:, :CH], cf[:, CH:]

    it = _make_iter_kernel()
    for k in range(K):
        if k < K - 1:
            gl, gh = it(gl, gh, srcI, dstI, a_mid, cml, cmh)
        else:
            gl, gh = it(gl, gh, srcI, dstI, a_fin, cfl, cfh)

    h = jnp.concatenate([gl[:N], gh[:N]], axis=1)
    return _mlp(h, W1, b1, W2, b2)
